# baseline (speedup 1.0000x reference)
import functools

import jax
import jax.numpy as jnp
from jax import lax
from jax.experimental import pallas as pl
from jax.experimental.pallas import tpu as pltpu
from jax.experimental.pallas import tpu_sc as plsc

B, L, D = 4096, 50, 128
LP = 128
LANES = 16
DV = D // LANES
NC, NS = 2, 16
NW = NC * NS
S_SC = 2560
RPW = S_SC // NW
CH = 8
NCHUNK = RPW // CH
NBUF = 2
RB = 128


def _sc_body(x_hbm, m_hbm, out_hbm, xbufs, mbufs, obufs, sems_in, sems_out):
    wid = lax.axis_index("s") * NC + lax.axis_index("c")
    base = wid * RPW

    def start_in(g, slot):
        rows = base + g * CH
        cx = pltpu.async_copy(
            x_hbm.at[:, pl.ds(rows, CH), :], xbufs[slot], sems_in[slot]
        )
        cm = pltpu.async_copy(m_hbm.at[pl.ds(rows, CH)], mbufs[slot], sems_in[slot])
        return (cx, cm)

    def compute(slot):
        xb, mb, ob = xbufs[slot], mbufs[slot], obufs[slot]

        nfull = L // LANES
        ntail = L - nfull * LANES

        def row_body(r, carry):
            del carry

            def grp_body(k, carry):
                accs = list(carry[:DV])
                cnt = carry[DV]
                mrow = mb[r, pl.ds(k * LANES, LANES)]
                for j in range(LANES):
                    m = mrow[j]
                    cnt = cnt + m
                    lpos = k * LANES + j
                    for d in range(DV):
                        accs[d] = accs[d] + xb[lpos, r, pl.ds(d * LANES, LANES)] * m
                return (*accs, cnt)

            init = tuple(jnp.zeros((LANES,), jnp.float32) for _ in range(DV + 1))
            res = lax.fori_loop(0, nfull, grp_body, init)
            accs = list(res[:DV])
            cnt = res[DV]
            mrow = mb[r, pl.ds(nfull * LANES, LANES)]
            for j in range(ntail):
                m = mrow[j]
                cnt = cnt + m
                for d in range(DV):
                    accs[d] = accs[d] + xb[nfull * LANES + j, r, pl.ds(d * LANES, LANES)] * m
            for d in range(DV):
                ob[r, pl.ds(d * LANES, LANES)] = accs[d] / cnt
            return 0

        lax.fori_loop(0, CH, row_body, 0)

    def start_out(g, slot):
        rows = base + g * CH
        return pltpu.async_copy(obufs[slot], out_hbm.at[pl.ds(rows, CH)], sems_out[slot])

    def wait_in(slot):
        pltpu.make_async_copy(
            x_hbm.at[:, pl.ds(0, CH), :], xbufs[slot], sems_in[slot]
        ).wait()
        pltpu.make_async_copy(m_hbm.at[pl.ds(0, CH)], mbufs[slot], sems_in[slot]).wait()

    def wait_out(slot):
        pltpu.make_async_copy(obufs[slot], out_hbm.at[pl.ds(0, CH)], sems_out[slot]).wait()

    for g in range(NBUF):
        start_in(g, g)

    def group_body(gg, carry):
        for b in range(NBUF):
            g = gg * NBUF + b
            wait_in(b)

            @pl.when(g >= NBUF)
            def _():
                wait_out(b)

            compute(b)
            start_out(g, b)

            @pl.when(g + NBUF < NCHUNK)
            def _():
                start_in(g + NBUF, b)

        return carry

    lax.fori_loop(0, NCHUNK // NBUF, group_body, 0)
    for b in range(NBUF):
        wait_out(b)


def _build_call():
    mesh = plsc.VectorSubcoreMesh(core_axis_name="c", subcore_axis_name="s")
    scratch = (
        [pltpu.VMEM((L, CH, D), jnp.float32) for _ in range(NBUF)],
        [pltpu.VMEM((CH, LP), jnp.float32) for _ in range(NBUF)],
        [pltpu.VMEM((CH, D), jnp.float32) for _ in range(NBUF)],
        [pltpu.SemaphoreType.DMA for _ in range(NBUF)],
        [pltpu.SemaphoreType.DMA for _ in range(NBUF)],
    )
    return pl.kernel(
        _sc_body,
        out_type=jax.ShapeDtypeStruct((S_SC, D), jnp.float32),
        mesh=mesh,
        scratch_types=scratch,
    )


_sc_call = _build_call()


def _tc_body(x_ref, m_ref, o_ref):
    acc = x_ref[0] * m_ref[:, 0:1]
    for l in range(1, L):
        acc = acc + x_ref[l] * m_ref[:, l : l + 1]
    cnt = jnp.sum(m_ref[:, :L], axis=1, keepdims=True)
    o_ref[...] = acc / cnt


def _build_tc_call():
    nblk = (B - S_SC) // RB
    off = S_SC // RB
    return pl.pallas_call(
        _tc_body,
        grid=(nblk,),
        in_specs=[
            pl.BlockSpec((L, RB, D), lambda i: (0, off + i, 0)),
            pl.BlockSpec((RB, LP), lambda i: (off + i, 0)),
        ],
        out_specs=pl.BlockSpec((RB, D), lambda i: (i, 0)),
        out_shape=jax.ShapeDtypeStruct((B - S_SC, D), jnp.float32),
    )


_tc_call = _build_tc_call()


@jax.jit
def kernel(inputs, mask):
    maskf = jnp.pad(mask.astype(jnp.float32), ((0, 0), (0, LP - L)))
    xt = jnp.transpose(inputs, (1, 0, 2))
    out_sc = _sc_call(xt, maskf)
    out_tc = _tc_call(xt, maskf)
    return jnp.concatenate([out_sc, out_tc], axis=0)


# speedup vs baseline: 1.0509x; 1.0509x over previous
import functools

import jax
import jax.numpy as jnp
from jax import lax
from jax.experimental import pallas as pl
from jax.experimental.pallas import tpu as pltpu
from jax.experimental.pallas import tpu_sc as plsc

B, L, D = 4096, 50, 128
LP = 128
LANES = 16
DV = D // LANES
NC, NS = 2, 16
NW = NC * NS
S_SC = 2048
RPW = S_SC // NW
CH = 8
NCHUNK = RPW // CH
NBUF = 2
RB = 128


def _sc_body(x_hbm, m_hbm, out_hbm, xbufs, mbufs, obufs, sems_in, sems_out):
    wid = lax.axis_index("s") * NC + lax.axis_index("c")
    base = wid * RPW

    def start_in(g, slot):
        rows = base + g * CH
        cx = pltpu.async_copy(
            x_hbm.at[:, pl.ds(rows, CH), :], xbufs[slot], sems_in[slot]
        )
        cm = pltpu.async_copy(m_hbm.at[pl.ds(rows, CH)], mbufs[slot], sems_in[slot])
        return (cx, cm)

    def compute(slot):
        xb, mb, ob = xbufs[slot], mbufs[slot], obufs[slot]

        nfull = L // LANES
        ntail = L - nfull * LANES

        def row_body(r, carry):
            del carry

            def grp_body(k, carry):
                accs = list(carry[:DV])
                cnt = carry[DV]
                mrow = mb[r, pl.ds(k * LANES, LANES)]
                for j in range(LANES):
                    m = mrow[j]
                    cnt = cnt + m
                    lpos = k * LANES + j
                    for d in range(DV):
                        accs[d] = accs[d] + xb[lpos, r, pl.ds(d * LANES, LANES)] * m
                return (*accs, cnt)

            init = tuple(jnp.zeros((LANES,), jnp.float32) for _ in range(DV + 1))
            res = lax.fori_loop(0, nfull, grp_body, init)
            accs = list(res[:DV])
            cnt = res[DV]
            mrow = mb[r, pl.ds(nfull * LANES, LANES)]
            for j in range(ntail):
                m = mrow[j]
                cnt = cnt + m
                for d in range(DV):
                    accs[d] = accs[d] + xb[nfull * LANES + j, r, pl.ds(d * LANES, LANES)] * m
            for d in range(DV):
                ob[r, pl.ds(d * LANES, LANES)] = accs[d] / cnt
            return 0

        lax.fori_loop(0, CH, row_body, 0)

    def start_out(g, slot):
        rows = base + g * CH
        return pltpu.async_copy(obufs[slot], out_hbm.at[pl.ds(rows, CH)], sems_out[slot])

    def wait_in(slot):
        pltpu.make_async_copy(
            x_hbm.at[:, pl.ds(0, CH), :], xbufs[slot], sems_in[slot]
        ).wait()
        pltpu.make_async_copy(m_hbm.at[pl.ds(0, CH)], mbufs[slot], sems_in[slot]).wait()

    def wait_out(slot):
        pltpu.make_async_copy(obufs[slot], out_hbm.at[pl.ds(0, CH)], sems_out[slot]).wait()

    for g in range(NBUF):
        start_in(g, g)

    def group_body(gg, carry):
        for b in range(NBUF):
            g = gg * NBUF + b
            wait_in(b)

            @pl.when(g >= NBUF)
            def _():
                wait_out(b)

            compute(b)
            start_out(g, b)

            @pl.when(g + NBUF < NCHUNK)
            def _():
                start_in(g + NBUF, b)

        return carry

    lax.fori_loop(0, NCHUNK // NBUF, group_body, 0)
    for b in range(NBUF):
        wait_out(b)


def _build_call():
    mesh = plsc.VectorSubcoreMesh(core_axis_name="c", subcore_axis_name="s")
    scratch = (
        [pltpu.VMEM((L, CH, D), jnp.float32) for _ in range(NBUF)],
        [pltpu.VMEM((CH, LP), jnp.float32) for _ in range(NBUF)],
        [pltpu.VMEM((CH, D), jnp.float32) for _ in range(NBUF)],
        [pltpu.SemaphoreType.DMA for _ in range(NBUF)],
        [pltpu.SemaphoreType.DMA for _ in range(NBUF)],
    )
    return pl.kernel(
        _sc_body,
        out_type=jax.ShapeDtypeStruct((S_SC, D), jnp.float32),
        mesh=mesh,
        scratch_types=scratch,
    )


_sc_call = _build_call()


def _tc_body(x_ref, m_ref, o_ref):
    acc = x_ref[0] * m_ref[:, 0:1]
    for l in range(1, L):
        acc = acc + x_ref[l] * m_ref[:, l : l + 1]
    cnt = jnp.sum(m_ref[:, :L], axis=1, keepdims=True)
    o_ref[...] = acc / cnt


def _build_tc_call():
    nblk = (B - S_SC) // RB
    off = S_SC // RB
    return pl.pallas_call(
        _tc_body,
        grid=(nblk,),
        in_specs=[
            pl.BlockSpec((L, RB, D), lambda i: (0, off + i, 0)),
            pl.BlockSpec((RB, LP), lambda i: (off + i, 0)),
        ],
        out_specs=pl.BlockSpec((RB, D), lambda i: (i, 0)),
        out_shape=jax.ShapeDtypeStruct((B - S_SC, D), jnp.float32),
    )


_tc_call = _build_tc_call()


@jax.jit
def kernel(inputs, mask):
    maskf = jnp.pad(mask.astype(jnp.float32), ((0, 0), (0, LP - L)))
    xt = jnp.transpose(inputs, (1, 0, 2))
    out_sc = _sc_call(xt, maskf)
    out_tc = _tc_call(xt, maskf)
    return jnp.concatenate([out_sc, out_tc], axis=0)
